# baseline (speedup 1.0000x reference)
import functools

import jax
import jax.numpy as jnp
from jax import lax
from jax.experimental import pallas as pl
from jax.experimental.pallas import tpu as pltpu
from jax.experimental.pallas import tpu_sc as plsc

N_NODES = 10000
HALF = 5000
PAD_ROWS = 56
NPAD = N_NODES + 2 * PAD_ROWS
D = 128
E_TOT = 330000
EH = E_TOT // 2
K = 128
NTILES = 16
NB = 81
LS = NB * K * NTILES
EPT = NB * K
RPT = NPAD // NTILES
MMB = NPAD // 16


NBUF = 3


def _sc_round_body(g_in, rc_buf, parts,
                   rc0, rc1, rc2,
                   rows0, rows1, rows2,
                   acc, sem0, sem1, sem2, sem3, sem4, sem5):
    k = lax.axis_index("c")
    t = lax.axis_index("s")
    rc = (rc0, rc1, rc2)
    rows = (rows0, rows1, rows2)
    sem = (sem0, sem1, sem2)
    sems = (sem3, sem4, sem5)

    pbase = (k * NTILES + t) * NB

    def start(pb, b):
        pltpu.sync_copy(rc_buf.at[pb], rc[b])
        pltpu.async_copy(g_in.at[rc[b].at[0]], rows[b], sem[b])

    for b in range(1, NBUF):
        start(pbase + b, b)

    zv = jnp.zeros((16,), jnp.float32)

    def zrow(i, c):
        for j in range(D // 16):
            rows0[i, pl.ds(j * 16, 16)] = zv
        return c

    lax.fori_loop(0, K, zrow, 0)
    zoff = 0
    while zoff < RPT:
        zn = min(K, RPT - zoff)
        pltpu.sync_copy(rows0.at[pl.ds(0, zn)],
                        acc.at[pl.ds(t * RPT + zoff, zn)])
        zoff += zn
    start(pbase, 0)
    plsc.subcore_barrier()


    def group(i, c):
        pb = pbase + NBUF * i
        for b in range(NBUF):
            pltpu.make_async_copy(g_in.at[rc[b].at[0]], rows[b], sem[b]).wait()
            pltpu.async_copy(rows[b], acc.at[rc[b].at[1]], sems[b], add=True)
        for b in range(NBUF):
            pltpu.make_async_copy(rows[b], acc.at[rc[b].at[1]], sems[b]).wait()

            @pl.when(i + 1 < NB // NBUF)
            def _():
                start(pb + b + NBUF, b)
        return c

    lax.fori_loop(0, NB // NBUF, group, 0)
    plsc.subcore_barrier()

    pltpu.sync_copy(acc.at[pl.ds(t * RPT, RPT)],
                    parts.at[pl.ds(k * NPAD + t * RPT, RPT)])


_sc_round = functools.partial(
    pl.kernel,
    out_type=jax.ShapeDtypeStruct((2 * NPAD, D), jnp.float32),
    mesh=plsc.VectorSubcoreMesh(core_axis_name="c", subcore_axis_name="s"),
    scratch_types=(
        [pltpu.VMEM((2, K), jnp.int32)] * 3
        + [pltpu.VMEM((K, D), jnp.float32)] * 3
        + [pltpu.VMEM_SHARED((NPAD, D), jnp.float32)]
        + [pltpu.SemaphoreType.DMA] * 6
    ),
)(_sc_round_body)


def _tc_comb_body(p0, p1, sb, ob):
    ob[...] = (p0[...] + p1[...]) * sb[...]


def _tc_comb(parts, sb):
    nblk = NPAD // MMB
    return pl.pallas_call(
        _tc_comb_body,
        grid=(nblk,),
        in_specs=[
            pl.BlockSpec((MMB, D), lambda i: (i, 0)),
            pl.BlockSpec((MMB, D), lambda i: (i + NPAD // MMB, 0)),
            pl.BlockSpec((MMB, D), lambda i: (i, 0)),
        ],
        out_specs=pl.BlockSpec((MMB, D), lambda i: (i, 0)),
        out_shape=jax.ShapeDtypeStruct((NPAD, D), jnp.float32),
    )(parts, parts, sb)


def _tc_a_body(xb, w1, b1, w2, b2, db, ob):
    h = jnp.dot(xb[...], w1[...], preferred_element_type=jnp.float32) + b1[...]
    h = jnp.maximum(h, 0.0)
    g = jnp.dot(h, w2[...], preferred_element_type=jnp.float32) + b2[...]
    ob[...] = g * db[...]


def _tc_b_body(hb, wc, bc, db, ob):
    z = jnp.dot(jnp.maximum(hb[...], 0.0), wc[...],
                preferred_element_type=jnp.float32) + bc[...]
    ob[...] = z * db[...]


def _tc_a(x_pad, w1, b1, w2, b2, dinv_b):
    return pl.pallas_call(
        _tc_a_body,
        grid=(NPAD // MMB,),
        in_specs=[
            pl.BlockSpec((MMB, D), lambda i: (i, 0)),
            pl.BlockSpec((D, D), lambda i: (0, 0)),
            pl.BlockSpec((1, D), lambda i: (0, 0)),
            pl.BlockSpec((D, D), lambda i: (0, 0)),
            pl.BlockSpec((1, D), lambda i: (0, 0)),
            pl.BlockSpec((MMB, D), lambda i: (i, 0)),
        ],
        out_specs=pl.BlockSpec((MMB, D), lambda i: (i, 0)),
        out_shape=jax.ShapeDtypeStruct((NPAD, D), jnp.float32),
    )(x_pad, w1, b1, w2, b2, dinv_b)


def _tc_b(h_pad, wc, bc, dinv_b):
    return pl.pallas_call(
        _tc_b_body,
        grid=(NPAD // MMB,),
        in_specs=[
            pl.BlockSpec((MMB, D), lambda i: (i, 0)),
            pl.BlockSpec((D, D), lambda i: (0, 0)),
            pl.BlockSpec((1, D), lambda i: (0, 0)),
            pl.BlockSpec((MMB, D), lambda i: (i, 0)),
        ],
        out_specs=pl.BlockSpec((MMB, D), lambda i: (i, 0)),
        out_shape=jax.ShapeDtypeStruct((NPAD, D), jnp.float32),
    )(h_pad, wc, bc, dinv_b)


def kernel(x, W1, b1, W2, b2, Wc, bc, edge_index, conv_time):
    ei = edge_index.astype(jnp.int32)
    ar = jnp.arange(N_NODES, dtype=jnp.int32)
    row = jnp.concatenate([ei[0], ar])
    col = jnp.concatenate([ei[1], ar])

    deg = jnp.zeros((N_NODES,), jnp.float32).at[col].add(1.0)
    dinv = jnp.where(deg > 0, lax.rsqrt(jnp.maximum(deg, 1e-12)), 0.0)
    dinv2 = dinv * dinv

    row_p = row + PAD_ROWS * (row >= HALF).astype(jnp.int32)
    col_p = col + PAD_ROWS * (col >= HALF).astype(jnp.int32)
    npad_e = LS - EH
    i_p = jnp.arange(npad_e, dtype=jnp.int32)
    fill_r = (i_p * 37) % 4096
    fill_c = HALF + (i_p % PAD_ROWS)
    row_buf = jnp.concatenate([row_p[:EH], fill_r, row_p[EH:], fill_r])
    col_buf = jnp.concatenate([col_p[:EH], fill_c, col_p[EH:], fill_c])
    rc_buf = jnp.stack([row_buf.reshape(-1, K), col_buf.reshape(-1, K)], axis=1)

    zpad = jnp.zeros((PAD_ROWS, D), jnp.float32)
    x_pad = jnp.concatenate([x[:HALF], zpad, x[HALF:], zpad])
    zp1 = jnp.zeros((PAD_ROWS,), jnp.float32)
    dinv_pad = jnp.concatenate([dinv[:HALF], zp1, dinv[HALF:], zp1])
    dinv2_pad = jnp.concatenate([dinv2[:HALF], zp1, dinv2[HALF:], zp1])
    dinv_b = jnp.broadcast_to(dinv_pad[:, None], (NPAD, D))
    dinv2_b = jnp.broadcast_to(dinv2_pad[:, None], (NPAD, D))

    g = _tc_a(x_pad, W1, b1.reshape(1, D), W2, b2.reshape(1, D), dinv_b)

    def round_(gg, sb):
        return _tc_comb(_sc_round(gg, rc_buf), sb)

    g = lax.fori_loop(0, conv_time - 1, lambda i, gg: round_(gg, dinv2_b), g)
    h = round_(g, dinv_b)

    ncls = Wc.shape[1]
    wc_pad = jnp.pad(Wc, ((0, 0), (0, D - ncls)))
    bc_pad = jnp.pad(bc, (0, D - ncls)).reshape(1, D)
    gc = _tc_b(h, wc_pad, bc_pad, dinv_b)
    outp = round_(gc, dinv_b)

    return jnp.concatenate([outp[:HALF, :ncls],
                            outp[HALF + PAD_ROWS:HALF + PAD_ROWS + HALF, :ncls]])


# speedup vs baseline: 1.0035x; 1.0035x over previous
import functools

import jax
import jax.numpy as jnp
from jax import lax
from jax.experimental import pallas as pl
from jax.experimental.pallas import tpu as pltpu
from jax.experimental.pallas import tpu_sc as plsc

N_NODES = 10000
HALF = 5000
PAD_ROWS = 56
NPAD = N_NODES + 2 * PAD_ROWS
D = 128
E_TOT = 330000
EH = E_TOT // 2
K = 128
NTILES = 16
NB = 81
LS = NB * K * NTILES
EPT = NB * K
RPT = NPAD // NTILES
MMB = NPAD // 16


NBUF = 3


def _sc_round_body(g_in, rc_buf, parts,
                   rc0, rc1, rc2,
                   rows0, rows1, rows2,
                   acc, sem0, sem1, sem2, sem3, sem4, sem5):
    k = lax.axis_index("c")
    t = lax.axis_index("s")
    rc = (rc0, rc1, rc2)
    rows = (rows0, rows1, rows2)
    sem = (sem0, sem1, sem2)
    sems = (sem3, sem4, sem5)

    zv = jnp.zeros((16,), jnp.float32)

    def zrow(i, c):
        for j in range(D // 16):
            rows0[i, pl.ds(j * 16, 16)] = zv
        return c

    lax.fori_loop(0, K, zrow, 0)
    zoff = 0
    while zoff < RPT:
        zn = min(K, RPT - zoff)
        pltpu.sync_copy(rows0.at[pl.ds(0, zn)],
                        acc.at[pl.ds(t * RPT + zoff, zn)])
        zoff += zn
    plsc.subcore_barrier()

    pbase = (k * NTILES + t) * NB

    def start(pb, b):
        pltpu.sync_copy(rc_buf.at[pb], rc[b])
        pltpu.async_copy(g_in.at[rc[b].at[0]], rows[b], sem[b])

    for b in range(NBUF):
        start(pbase + b, b)

    def group(i, c):
        pb = pbase + NBUF * i
        for b in range(NBUF):
            pltpu.make_async_copy(g_in.at[rc[b].at[0]], rows[b], sem[b]).wait()
            pltpu.async_copy(rows[b], acc.at[rc[b].at[1]], sems[b], add=True)
        for b in range(NBUF):
            pltpu.make_async_copy(rows[b], acc.at[rc[b].at[1]], sems[b]).wait()

            @pl.when(i + 1 < NB // NBUF)
            def _():
                start(pb + b + NBUF, b)
        return c

    lax.fori_loop(0, NB // NBUF, group, 0)
    plsc.subcore_barrier()

    pltpu.sync_copy(acc.at[pl.ds(t * RPT, RPT)],
                    parts.at[pl.ds(k * NPAD + t * RPT, RPT)])


_sc_round = functools.partial(
    pl.kernel,
    out_type=jax.ShapeDtypeStruct((2 * NPAD, D), jnp.float32),
    mesh=plsc.VectorSubcoreMesh(core_axis_name="c", subcore_axis_name="s"),
    scratch_types=(
        [pltpu.VMEM((2, K), jnp.int32)] * 3
        + [pltpu.VMEM((K, D), jnp.float32)] * 3
        + [pltpu.VMEM_SHARED((NPAD, D), jnp.float32)]
        + [pltpu.SemaphoreType.DMA] * 6
    ),
)(_sc_round_body)


def _tc_comb_body(p0, p1, sb, ob):
    ob[...] = (p0[...] + p1[...]) * sb[...]


def _tc_comb(parts, sb):
    nblk = NPAD // MMB
    return pl.pallas_call(
        _tc_comb_body,
        grid=(nblk,),
        in_specs=[
            pl.BlockSpec((MMB, D), lambda i: (i, 0)),
            pl.BlockSpec((MMB, D), lambda i: (i + NPAD // MMB, 0)),
            pl.BlockSpec((MMB, D), lambda i: (i, 0)),
        ],
        out_specs=pl.BlockSpec((MMB, D), lambda i: (i, 0)),
        out_shape=jax.ShapeDtypeStruct((NPAD, D), jnp.float32),
    )(parts, parts, sb)


def _tc_a_body(xb, w1, b1, w2, b2, db, ob):
    h = jnp.dot(xb[...], w1[...], preferred_element_type=jnp.float32) + b1[...]
    h = jnp.maximum(h, 0.0)
    g = jnp.dot(h, w2[...], preferred_element_type=jnp.float32) + b2[...]
    ob[...] = g * db[...]


def _tc_b_body(hb, wc, bc, db, ob):
    z = jnp.dot(jnp.maximum(hb[...], 0.0), wc[...],
                preferred_element_type=jnp.float32) + bc[...]
    ob[...] = z * db[...]


def _tc_a(x_pad, w1, b1, w2, b2, dinv_b):
    return pl.pallas_call(
        _tc_a_body,
        grid=(NPAD // MMB,),
        in_specs=[
            pl.BlockSpec((MMB, D), lambda i: (i, 0)),
            pl.BlockSpec((D, D), lambda i: (0, 0)),
            pl.BlockSpec((1, D), lambda i: (0, 0)),
            pl.BlockSpec((D, D), lambda i: (0, 0)),
            pl.BlockSpec((1, D), lambda i: (0, 0)),
            pl.BlockSpec((MMB, D), lambda i: (i, 0)),
        ],
        out_specs=pl.BlockSpec((MMB, D), lambda i: (i, 0)),
        out_shape=jax.ShapeDtypeStruct((NPAD, D), jnp.float32),
    )(x_pad, w1, b1, w2, b2, dinv_b)


def _tc_b(h_pad, wc, bc, dinv_b):
    return pl.pallas_call(
        _tc_b_body,
        grid=(NPAD // MMB,),
        in_specs=[
            pl.BlockSpec((MMB, D), lambda i: (i, 0)),
            pl.BlockSpec((D, D), lambda i: (0, 0)),
            pl.BlockSpec((1, D), lambda i: (0, 0)),
            pl.BlockSpec((MMB, D), lambda i: (i, 0)),
        ],
        out_specs=pl.BlockSpec((MMB, D), lambda i: (i, 0)),
        out_shape=jax.ShapeDtypeStruct((NPAD, D), jnp.float32),
    )(h_pad, wc, bc, dinv_b)


def kernel(x, W1, b1, W2, b2, Wc, bc, edge_index, conv_time):
    ei = edge_index.astype(jnp.int32)
    ar = jnp.arange(N_NODES, dtype=jnp.int32)
    row = jnp.concatenate([ei[0], ar])
    col = jnp.concatenate([ei[1], ar])

    deg = jnp.zeros((N_NODES,), jnp.float32).at[col].add(1.0)
    dinv = jnp.where(deg > 0, lax.rsqrt(jnp.maximum(deg, 1e-12)), 0.0)
    dinv2 = dinv * dinv

    row_p = row + PAD_ROWS * (row >= HALF).astype(jnp.int32)
    col_p = col + PAD_ROWS * (col >= HALF).astype(jnp.int32)
    npad_e = LS - EH
    i_p = jnp.arange(npad_e, dtype=jnp.int32)
    fill_r = (i_p * 37) % 4096
    fill_c = HALF + (i_p % PAD_ROWS)
    row_buf = jnp.concatenate([row_p[:EH], fill_r, row_p[EH:], fill_r])
    col_buf = jnp.concatenate([col_p[:EH], fill_c, col_p[EH:], fill_c])
    rc_buf = jnp.stack([row_buf.reshape(-1, K), col_buf.reshape(-1, K)], axis=1)

    zpad = jnp.zeros((PAD_ROWS, D), jnp.float32)
    x_pad = jnp.concatenate([x[:HALF], zpad, x[HALF:], zpad])
    zp1 = jnp.zeros((PAD_ROWS,), jnp.float32)
    dinv_pad = jnp.concatenate([dinv[:HALF], zp1, dinv[HALF:], zp1])
    dinv2_pad = jnp.concatenate([dinv2[:HALF], zp1, dinv2[HALF:], zp1])
    dinv_b = jnp.broadcast_to(dinv_pad[:, None], (NPAD, D))
    dinv2_b = jnp.broadcast_to(dinv2_pad[:, None], (NPAD, D))

    g = _tc_a(x_pad, W1, b1.reshape(1, D), W2, b2.reshape(1, D), dinv_b)

    def round_(gg, sb):
        return _tc_comb(_sc_round(gg, rc_buf), sb)

    g = lax.fori_loop(0, conv_time - 1, lambda i, gg: round_(gg, dinv2_b), g)
    h = round_(g, dinv_b)

    ncls = Wc.shape[1]
    wc_pad = jnp.pad(Wc, ((0, 0), (0, D - ncls)))
    bc_pad = jnp.pad(bc, (0, D - ncls)).reshape(1, D)
    gc = _tc_b(h, wc_pad, bc_pad, dinv_b)
    outp = round_(gc, dinv_b)

    return jnp.concatenate([outp[:HALF, :ncls],
                            outp[HALF + PAD_ROWS:HALF + PAD_ROWS + HALF, :ncls]])
